# Initial kernel scaffold; baseline (speedup 1.0000x reference)
import jax
import jax.numpy as jnp
from jax.experimental import pallas as pl


def kernel(ref_points_c, src_points_c, gt_node_corr_overlaps, gt_node_corr_indices, ref_node_corr_indices, src_node_corr_indices, ref_corr_points, src_corr_points, transform, estimated_transform):
    raise NotImplementedError("write your pallas kernel here")


# speedup vs baseline: 1.8373x; 1.8373x over previous
import functools
import math

import jax
import jax.numpy as jnp
from jax import lax
from jax.experimental import pallas as pl
from jax.experimental.pallas import tpu as pltpu
from jax.experimental.pallas import tpu_sc as plsc

_NROWS = 2048
_NCOLS = 2048
_CHUNK_ROWS = 32
_NUM_WORKERS = 32
_PPAD = 30720


def _sc_body(gt_ref_h, gt_src_h, ovl_h, qr_h, qs_h,
             rx_h, ry_h, rz_h, sx_h, sy_h, sz_h, cst_h,
             c_out_h, f_out_h,
             gt_ref_v, gt_src_v, ovl_v, qr_v, qs_v, mapb,
             rx_v, ry_v, rz_v, sx_v, sy_v, sz_v, cst_v,
             acc_c_v, acc_f_v):
    wid = lax.axis_index("s") * 2 + lax.axis_index("c")

    pltpu.sync_copy(gt_ref_h, gt_ref_v)
    pltpu.sync_copy(gt_src_h, gt_src_v)
    pltpu.sync_copy(ovl_h, ovl_v)
    pltpu.sync_copy(qr_h, qr_v)
    pltpu.sync_copy(qs_h, qs_v)
    ppw = _PPAD // _NUM_WORKERS
    pbase = wid * ppw
    pltpu.sync_copy(rx_h.at[pl.ds(pbase, ppw)], rx_v)
    pltpu.sync_copy(ry_h.at[pl.ds(pbase, ppw)], ry_v)
    pltpu.sync_copy(rz_h.at[pl.ds(pbase, ppw)], rz_v)
    pltpu.sync_copy(sx_h.at[pl.ds(pbase, ppw)], sx_v)
    pltpu.sync_copy(sy_h.at[pl.ds(pbase, ppw)], sy_v)
    pltpu.sync_copy(sz_h.at[pl.ds(pbase, ppw)], sz_v)
    pltpu.sync_copy(cst_h, cst_v)

    ones = jnp.ones((16,), jnp.float32)
    zeros = jnp.zeros((16,), jnp.float32)
    lane = lax.iota(jnp.int32, 16)

    r00 = cst_v[0, :]; r01 = cst_v[1, :]; r02 = cst_v[2, :]
    r10 = cst_v[3, :]; r11 = cst_v[4, :]; r12 = cst_v[5, :]
    r20 = cst_v[6, :]; r21 = cst_v[7, :]; r22 = cst_v[8, :]
    t0 = cst_v[9, :]; t1 = cst_v[10, :]; t2 = cst_v[11, :]

    def fbody(i, acc):
        o = i * 16
        vx = sx_v[pl.ds(o, 16)]
        vy = sy_v[pl.ds(o, 16)]
        vz = sz_v[pl.ds(o, 16)]
        tx = r00 * vx + r01 * vy + r02 * vz + t0
        ty = r10 * vx + r11 * vy + r12 * vz + t1
        tz = r20 * vx + r21 * vy + r22 * vz + t2
        dx = rx_v[pl.ds(o, 16)] - tx
        dy = ry_v[pl.ds(o, 16)] - ty
        dz = rz_v[pl.ds(o, 16)] - tz
        d2 = dx * dx + dy * dy + dz * dz
        gidx = pbase + o + lane
        m = (gidx < 30000) & (d2 < 0.01)
        return acc + jnp.where(m, 1.0, 0.0)

    facc = lax.fori_loop(0, ppw // 16, fbody, zeros)

    n_entry_vecs = gt_ref_v.shape[0] // 16
    n_query_vecs = qr_v.shape[0] // 16
    cacc = zeros
    for half in range(2):
        cbase = (wid + _NUM_WORKERS * half) * _CHUNK_ROWS

        def zbody(i, _):
            qr = qr_v[pl.ds(i * 16, 16)]
            qs = qs_v[pl.ds(i * 16, 16)]
            rel = qr - cbase
            m = (rel >= 0) & (rel < _CHUNK_ROWS)
            idx = jnp.where(m, rel * _NCOLS + qs, 0)
            plsc.store_scatter(mapb, [idx], zeros, mask=m)
            return 0

        lax.fori_loop(0, n_query_vecs, zbody, 0)

        def sbody(i, _):
            er = gt_ref_v[pl.ds(i * 16, 16)]
            es = gt_src_v[pl.ds(i * 16, 16)]
            eo = ovl_v[pl.ds(i * 16, 16)]
            rel = er - cbase
            m = (rel >= 0) & (rel < _CHUNK_ROWS) & (eo > 0.0)
            idx = jnp.where(m, rel * _NCOLS + es, 0)
            plsc.store_scatter(mapb, [idx], ones, mask=m)
            return 0

        lax.fori_loop(0, n_entry_vecs, sbody, 0)

        def gbody(i, acc):
            qr = qr_v[pl.ds(i * 16, 16)]
            qs = qs_v[pl.ds(i * 16, 16)]
            rel = qr - cbase
            m = (rel >= 0) & (rel < _CHUNK_ROWS)
            idx = jnp.where(m, rel * _NCOLS + qs, 0)
            v = plsc.load_gather(mapb, [idx], mask=m)
            return acc + jnp.where(m, v, 0.0)

        cacc = lax.fori_loop(0, n_query_vecs, gbody, cacc)

    acc_c_v[...] = cacc
    acc_f_v[...] = facc
    pltpu.sync_copy(acc_c_v, c_out_h.at[wid])
    pltpu.sync_copy(acc_f_v, f_out_h.at[wid])


def _tc_body(cpart_ref, fpart_ref, t_ref, e_ref, out_ref):
    c_prec = jnp.sum(cpart_ref[...]) * (1.0 / 4096.0)
    f_prec = jnp.sum(fpart_ref[...]) * (1.0 / 30000.0)
    t = t_ref[...]
    e = e_ref[...]
    tr = jnp.sum(t[:3, :3] * e[:3, :3])
    x = jnp.clip((tr - 1.0) * 0.5, -1.0, 1.0)
    acos = jnp.arctan2(jnp.sqrt(jnp.maximum(1.0 - x * x, 0.0)), x)
    rre = acos * (180.0 / math.pi)
    dt = t[:3, 3] - e[:3, 3]
    rte = jnp.sqrt(jnp.sum(dt * dt))
    recall = jnp.where((rre < 15.0) & (rte < 0.3), 1.0, 0.0)
    i8 = lax.broadcasted_iota(jnp.int32, (1, 8), 1)
    v = jnp.where(i8 == 0, c_prec,
        jnp.where(i8 == 1, f_prec,
        jnp.where(i8 == 2, rre,
        jnp.where(i8 == 3, rte,
        jnp.where(i8 == 4, recall, 0.0)))))
    out_ref[...] = v


def kernel(ref_points_c, src_points_c, gt_node_corr_overlaps, gt_node_corr_indices,
           ref_node_corr_indices, src_node_corr_indices, ref_corr_points,
           src_corr_points, transform, estimated_transform):
    gt_ref = gt_node_corr_indices[:, 0].astype(jnp.int32)
    gt_src = gt_node_corr_indices[:, 1].astype(jnp.int32)
    qr = ref_node_corr_indices.astype(jnp.int32)
    qs = src_node_corr_indices.astype(jnp.int32)
    p = ref_corr_points.shape[0]
    rt = jnp.pad(ref_corr_points, ((0, _PPAD - p), (0, 0))).T
    st = jnp.pad(src_corr_points, ((0, _PPAD - p), (0, 0))).T
    cvals = jnp.concatenate([transform[:3, :3].reshape(-1), transform[:3, 3]])
    cst = jnp.broadcast_to(cvals[:, None], (12, 16)).astype(jnp.float32)

    mesh = plsc.VectorSubcoreMesh(core_axis_name="c", subcore_axis_name="s",
                                  num_cores=2, num_subcores=16)
    m = gt_ref.shape[0]
    k = qr.shape[0]
    ppw = _PPAD // _NUM_WORKERS
    sc_fn = functools.partial(
        pl.kernel,
        out_type=[
            jax.ShapeDtypeStruct((_NUM_WORKERS, 16), jnp.float32),
            jax.ShapeDtypeStruct((_NUM_WORKERS, 16), jnp.float32),
        ],
        mesh=mesh,
        scratch_types=[
            pltpu.VMEM((m,), jnp.int32),
            pltpu.VMEM((m,), jnp.int32),
            pltpu.VMEM((m,), jnp.float32),
            pltpu.VMEM((k,), jnp.int32),
            pltpu.VMEM((k,), jnp.int32),
            pltpu.VMEM((_CHUNK_ROWS * _NCOLS,), jnp.float32),
            pltpu.VMEM((ppw,), jnp.float32),
            pltpu.VMEM((ppw,), jnp.float32),
            pltpu.VMEM((ppw,), jnp.float32),
            pltpu.VMEM((ppw,), jnp.float32),
            pltpu.VMEM((ppw,), jnp.float32),
            pltpu.VMEM((ppw,), jnp.float32),
            pltpu.VMEM((12, 16), jnp.float32),
            pltpu.VMEM((16,), jnp.float32),
            pltpu.VMEM((16,), jnp.float32),
        ],
        compiler_params=pltpu.CompilerParams(needs_layout_passes=False),
    )(_sc_body)
    c_part, f_part = sc_fn(gt_ref, gt_src, gt_node_corr_overlaps, qr, qs,
                           rt[0], rt[1], rt[2], st[0], st[1], st[2], cst)

    res = pl.pallas_call(
        _tc_body,
        out_shape=jax.ShapeDtypeStruct((1, 8), jnp.float32),
    )(c_part, f_part, transform.astype(jnp.float32),
      estimated_transform.astype(jnp.float32))
    return res[0, :5]
